# Initial kernel scaffold; baseline (speedup 1.0000x reference)
import jax
import jax.numpy as jnp
from jax.experimental import pallas as pl


def kernel(index, table):
    raise NotImplementedError("write your pallas kernel here")


# speedup vs baseline: 4.9130x; 4.9130x over previous
import functools

import jax
import jax.numpy as jnp
from jax import lax
from jax.experimental import pallas as pl
from jax.experimental.pallas import tpu as pltpu
from jax.experimental.pallas import tpu_sc as plsc

L = 16
NC = 2
NS = 16
NW = NC * NS


def _sc_gather(idx_flat, table_flat, n_rows, d):
    n = idx_flat.shape[0]
    npw = n // NW
    c = 2048
    nchunk = npw // c
    assert npw % c == 0 and c % L == 0

    mesh = plsc.VectorSubcoreMesh(core_axis_name="c", subcore_axis_name="s")

    @functools.partial(
        pl.kernel,
        out_type=jax.ShapeDtypeStruct((n * d,), jnp.float32),
        mesh=mesh,
        scratch_types=[
            pltpu.VMEM((n_rows * d,), jnp.float32),
            pltpu.VMEM((c,), jnp.int32),
            pltpu.VMEM((c * d,), jnp.float32),
        ],
        compiler_params=pltpu.CompilerParams(needs_layout_passes=False),
    )
    def k(idx_hbm, table_hbm, out_hbm, table_v, idx_v, out_v):
        wid = lax.axis_index("s") * NC + lax.axis_index("c")
        base = wid * npw
        pltpu.sync_copy(table_hbm, table_v)
        lane_d = lax.iota(jnp.int32, L) * d

        @pl.loop(0, nchunk)
        def chunk_body(ci):
            off = base + ci * c
            pltpu.sync_copy(idx_hbm.at[pl.ds(off, c)], idx_v)

            @pl.loop(0, c // L)
            def vec_body(kk):
                idxv = idx_v[pl.ds(kk * L, L)]
                a0 = idxv * d
                p0 = lane_d + kk * (L * d)
                for dd in range(d):
                    v = plsc.load_gather(table_v, [a0 + dd])
                    plsc.store_scatter(out_v, [p0 + dd], v)

            pltpu.sync_copy(out_v, out_hbm.at[pl.ds(off * d, c * d)])

    return k(idx_flat, table_flat)


def kernel(index, table):
    n_rows, d = table.shape
    idx_flat = index.reshape(-1)
    out_flat = _sc_gather(idx_flat, table.reshape(-1), n_rows, d)
    return out_flat.reshape(index.shape + (d,))


# speedup vs baseline: 5.3718x; 1.0934x over previous
import functools

import jax
import jax.numpy as jnp
from jax import lax
from jax.experimental import pallas as pl
from jax.experimental.pallas import tpu as pltpu
from jax.experimental.pallas import tpu_sc as plsc

L = 16
NC = 2
NS = 16
NW = NC * NS


def _sc_gather(idx_flat, table_flat, n_rows, d):
    n = idx_flat.shape[0]
    npw = n // NW
    c = 2048
    nchunk = npw // c
    assert npw % c == 0 and c % L == 0

    mesh = plsc.VectorSubcoreMesh(core_axis_name="c", subcore_axis_name="s")

    @functools.partial(
        pl.kernel,
        out_type=jax.ShapeDtypeStruct((n * d,), jnp.float32),
        mesh=mesh,
        scratch_types=[
            pltpu.VMEM((n_rows * d,), jnp.float32),
            pltpu.VMEM((c,), jnp.int32),
            pltpu.VMEM((c * d,), jnp.float32),
        ],
        compiler_params=pltpu.CompilerParams(needs_layout_passes=False),
    )
    def k(idx_hbm, table_hbm, out_hbm, table_v, idx_v, out_v):
        wid = lax.axis_index("s") * NC + lax.axis_index("c")
        base = wid * npw
        pltpu.sync_copy(table_hbm, table_v)
        lane_d = lax.iota(jnp.int32, L) * d

        @pl.loop(0, nchunk)
        def chunk_body(ci):
            off = base + ci * c
            pltpu.sync_copy(idx_hbm.at[pl.ds(off, c)], idx_v)

            @plsc.parallel_loop(0, c // L, unroll=8)
            def vec_body(kk):
                idxv = idx_v[pl.ds(kk * L, L)]
                a0 = idxv * d
                p0 = lane_d + kk * (L * d)
                for dd in range(d):
                    v = plsc.load_gather(table_v, [a0 + dd])
                    plsc.store_scatter(out_v, [p0 + dd], v)

            pltpu.sync_copy(out_v, out_hbm.at[pl.ds(off * d, c * d)])

    return k(idx_flat, table_flat)


def kernel(index, table):
    n_rows, d = table.shape
    idx_flat = index.reshape(-1)
    out_flat = _sc_gather(idx_flat, table.reshape(-1), n_rows, d)
    return out_flat.reshape(index.shape + (d,))


# speedup vs baseline: 92.8067x; 17.2767x over previous
import functools

import jax
import jax.numpy as jnp
from jax import lax
from jax.experimental import pallas as pl
from jax.experimental.pallas import tpu as pltpu
from jax.experimental.pallas import tpu_sc as plsc

L = 16
NC = 2
NS = 16
NW = NC * NS


def _sc_gather_t(idx_t, table_flat, n_rows, d):
    rows, cols = idx_t.shape
    cw = cols // NW
    rb = 8
    nblk = rows // rb
    assert rows % rb == 0 and cols % NW == 0 and cw % L == 0

    mesh = plsc.VectorSubcoreMesh(core_axis_name="c", subcore_axis_name="s")

    @functools.partial(
        pl.kernel,
        out_type=jax.ShapeDtypeStruct((d, rows, cols), jnp.float32),
        mesh=mesh,
        scratch_types=[
            pltpu.VMEM((n_rows * d,), jnp.float32),
            pltpu.VMEM((rb, cw), jnp.int32),
            pltpu.VMEM((d, rb, cw), jnp.float32),
        ],
        compiler_params=pltpu.CompilerParams(
            needs_layout_passes=False,
            use_tc_tiling_on_sc=True,
        ),
    )
    def k(idx_hbm, table_hbm, out_hbm, table_v, idx_v, out_v):
        wid = lax.axis_index("s") * NC + lax.axis_index("c")
        i0 = wid * cw
        pltpu.sync_copy(table_hbm, table_v)
        nvec = cw // L

        @pl.loop(0, nblk)
        def blk_body(bi):
            j0 = bi * rb
            pltpu.sync_copy(idx_hbm.at[pl.ds(j0, rb), pl.ds(i0, cw)], idx_v)

            @plsc.parallel_loop(0, rb * nvec, unroll=4)
            def vec_body(m):
                jj = m // nvec
                kk = (m % nvec) * L
                idxv = idx_v[jj, pl.ds(kk, L)]
                a0 = idxv * d
                for dd in range(d):
                    v = plsc.load_gather(table_v, [a0 + dd])
                    out_v[dd, jj, pl.ds(kk, L)] = v

            for dd in range(d):
                pltpu.sync_copy(
                    out_v.at[dd],
                    out_hbm.at[dd, pl.ds(j0, rb), pl.ds(i0, cw)],
                )

    return k(idx_t, table_flat)


def kernel(index, table):
    n_rows, d = table.shape
    out_t = _sc_gather_t(index.T, table.reshape(-1), n_rows, d)
    return out_t.transpose(2, 1, 0)


# speedup vs baseline: 162.8533x; 1.7548x over previous
import functools

import jax
import jax.numpy as jnp
from jax import lax
from jax.experimental import pallas as pl
from jax.experimental.pallas import tpu as pltpu
from jax.experimental.pallas import tpu_sc as plsc

L = 16
NC = 2
NS = 16
NW = NC * NS


def _sc_gather_t(idx_t, table_flat, n_rows, d):
    rows, cols = idx_t.shape
    cw = cols // NW
    rb = 8
    hw = cw // 2
    nblk = rows // rb
    total = 2 * nblk
    assert rows % rb == 0 and cols % NW == 0 and hw % L == 0

    mesh = plsc.VectorSubcoreMesh(core_axis_name="c", subcore_axis_name="s")

    @functools.partial(
        pl.kernel,
        out_type=jax.ShapeDtypeStruct((d, rows, cols), jnp.float32),
        mesh=mesh,
        scratch_types=[
            pltpu.VMEM((n_rows * d,), jnp.float32),
            pltpu.VMEM((rb, hw), jnp.int32),
            pltpu.VMEM((rb, hw), jnp.int32),
            pltpu.VMEM((d, rb, hw), jnp.float32),
            pltpu.VMEM((d, rb, hw), jnp.float32),
            pltpu.SemaphoreType.DMA,
            pltpu.SemaphoreType.DMA,
            pltpu.SemaphoreType.DMA,
            pltpu.SemaphoreType.DMA,
        ],
        compiler_params=pltpu.CompilerParams(
            needs_layout_passes=False,
            use_tc_tiling_on_sc=True,
        ),
    )
    def k(idx_hbm, table_hbm, out_hbm, table_v,
          idx_v0, idx_v1, out_v0, out_v1, si0, si1, so0, so1):
        wid = lax.axis_index("s") * NC + lax.axis_index("c")
        i0 = wid * cw
        idx_vs, out_vs = (idx_v0, idx_v1), (out_v0, out_v1)
        sis, sos = (si0, si1), (so0, so1)
        pltpu.sync_copy(table_hbm, table_v)
        nvec = hw // L

        def idx_src(m):
            return idx_hbm.at[pl.ds((m // 2) * rb, rb),
                              pl.ds(i0 + (m % 2) * hw, hw)]

        def issue_idx(m, b):
            return pltpu.async_copy(idx_src(m), idx_vs[b], sis[b])

        def wait_idx(m, b):
            pltpu.make_async_copy(idx_src(m), idx_vs[b], sis[b]).wait()

        def drain_out(b):
            pltpu.make_async_copy(
                out_hbm.at[pl.ds(0, d), pl.ds(0, rb), pl.ds(0, hw)],
                out_vs[b], sos[b]).wait()

        def compute(b):
            @plsc.parallel_loop(0, rb * nvec, unroll=4)
            def vec_body(m):
                jj = m // nvec
                kk = (m % nvec) * L
                idxv = idx_vs[b][jj, pl.ds(kk, L)]
                a0 = idxv * d
                for dd in range(d):
                    v = plsc.load_gather(table_v, [a0 + dd])
                    out_vs[b][dd, jj, pl.ds(kk, L)] = v

        def issue_out(m, b):
            j0 = (m // 2) * rb
            ic = i0 + (m % 2) * hw
            for dd in range(d):
                pltpu.async_copy(out_vs[b].at[dd],
                                 out_hbm.at[dd, pl.ds(j0, rb), pl.ds(ic, hw)],
                                 sos[b])

        issue_idx(0, 0)
        issue_idx(1, 1)

        for m in range(2):
            wait_idx(m, m)
            compute(m)
            issue_out(m, m)
            issue_idx(m + 2, m)

        @pl.loop(2, total, step=2)
        def ring(t):
            for b in range(2):
                m = t + b
                wait_idx(m, b)
                drain_out(b)
                compute(b)
                issue_out(m, b)

                @pl.when(m + 2 < total)
                def _():
                    issue_idx(m + 2, b)

        drain_out(0)
        drain_out(1)

    return k(idx_t, table_flat)


def kernel(index, table):
    n_rows, d = table.shape
    out_t = _sc_gather_t(index.T, table.reshape(-1), n_rows, d)
    return out_t.transpose(2, 1, 0)
